# baseline (speedup 1.0000x reference)
import functools

import jax
import jax.numpy as jnp
from jax import lax
from jax.experimental import pallas as pl
from jax.experimental.pallas import tpu as pltpu
from jax.experimental.pallas import tpu_sc as plsc

B = 16384
D = 32
NR = 1000000
SR = NR // 4
NW = 32
BPW = B // NW
PK = BPW // 4
RPW = NR // NW
CHR = 320
ACH = CHR // 4
NIT = 98

GCH = 128
NGCH = BPW // GCH


@functools.cache
def _build_repack():
    mesh = plsc.VectorSubcoreMesh(core_axis_name="c", subcore_axis_name="s")

    @functools.partial(
        pl.kernel,
        mesh=mesh,
        out_type=tuple(
            jax.ShapeDtypeStruct((SR, 128), jnp.float32) for _ in range(4)),
        scratch_types=(
            pltpu.VMEM((2, CHR, D), jnp.float32),
            pltpu.VMEM((2, ACH, 128), jnp.float32),
            pltpu.SemaphoreType.DMA,
            pltpu.SemaphoreType.DMA,
        ),
    )
    def _repack(t0, t1, t2, t3, o0, o1, o2, o3, vbuf, asm, sem_i, sem_o):
        wid = lax.axis_index("s") * 2 + lax.axis_index("c")
        astart = ((wid * RPW + 31) // 32) * 32

        def cs(t):
            return pl.multiple_of(
                jnp.minimum(astart + t * CHR, NR - CHR), 32)

        for tbl, out in ((t0, o0), (t1, o1), (t2, o2), (t3, o3)):

            def fire_in(t):
                pltpu.async_copy(
                    tbl.at[pl.ds(cs(t), CHR)], vbuf.at[t % 2], sem_i)

            def wait_in():
                pltpu.make_async_copy(
                    tbl.at[pl.ds(0, CHR)], vbuf.at[0], sem_i).wait()

            def wait_out():
                pltpu.make_async_copy(
                    out.at[pl.ds(0, ACH)], asm.at[0], sem_o).wait()

            fire_in(0)

            def body(t, carry):
                p = t % 2
                fire_in(t + 1)

                @pl.when(t > 0)
                def _():
                    wait_out()

                wait_in()

                def asm_body(i4, c2):
                    for k in range(4):
                        r = i4 * 4 + k
                        asm[p, i4, pl.ds(k * D, 16)] = vbuf[p, r,
                                                            pl.ds(0, 16)]
                        asm[p, i4, pl.ds(k * D + 16, 16)] = vbuf[
                            p, r, pl.ds(16, 16)]
                    return c2

                lax.fori_loop(0, ACH, asm_body, 0)
                pltpu.async_copy(
                    asm.at[p], out.at[pl.ds(pl.multiple_of(cs(t) // 4, 8), ACH)], sem_o)
                return carry

            lax.fori_loop(0, NIT - 1, body, 0)
            p = (NIT - 1) % 2
            wait_out()
            wait_in()

            def asm_tail(i4, c2):
                for k in range(4):
                    r = i4 * 4 + k
                    asm[p, i4, pl.ds(k * D, 16)] = vbuf[p, r, pl.ds(0, 16)]
                    asm[p, i4, pl.ds(k * D + 16, 16)] = vbuf[p, r,
                                                             pl.ds(16, 16)]
                return c2

            lax.fori_loop(0, ACH, asm_tail, 0)
            pltpu.async_copy(
                asm.at[p], out.at[pl.ds(pl.multiple_of(cs(NIT - 1) // 4, 8), ACH)], sem_o)
            wait_out()

    return _repack


@functools.cache
def _build_sc_gather():
    mesh = plsc.VectorSubcoreMesh(core_axis_name="c", subcore_axis_name="s")

    @functools.partial(
        pl.kernel,
        mesh=mesh,
        out_type=(
            jax.ShapeDtypeStruct((B // 4, 128), jnp.float32),
            jax.ShapeDtypeStruct((B // 4, 128), jnp.float32),
            jax.ShapeDtypeStruct((B // 4, 128), jnp.float32),
        ),
        scratch_types=(
            pltpu.VMEM((BPW,), jnp.int32),
            pltpu.VMEM((BPW,), jnp.int32),
            pltpu.VMEM((BPW,), jnp.int32),
            pltpu.VMEM((BPW,), jnp.int32),
            pltpu.VMEM((2, GCH, 128), jnp.float32),
            pltpu.VMEM((PK, 128), jnp.float32),
            pltpu.VMEM((PK, 128), jnp.float32),
            pltpu.VMEM((PK, 128), jnp.float32),
            pltpu.SemaphoreType.DMA,
            pltpu.SemaphoreType.DMA,
        ),
    )
    def _sc_gather(uq_hbm, iq_hbm, ul_hbm, il_hbm,
                   umf_hbm, imf_hbm, umlp_hbm, imlp_hbm,
                   mf_out, umlp_out, imlp_out,
                   uq_v, iq_v, ul_v, il_v, gbuf, pk_a, pk_c, pk_d,
                   sem0, sem1):
        wid = lax.axis_index("s") * 2 + lax.axis_index("c")
        pbase = wid * PK
        sems = (sem0, sem1)

        pltpu.sync_copy(uq_hbm.at[wid], uq_v)
        pltpu.sync_copy(iq_hbm.at[wid], iq_v)
        pltpu.sync_copy(ul_hbm.at[wid], ul_v)
        pltpu.sync_copy(il_hbm.at[wid], il_v)

        plan = ((umf_hbm, uq_v, ul_v, pk_a, False),
                (imf_hbm, iq_v, il_v, pk_a, True),
                (umlp_hbm, uq_v, ul_v, pk_c, False),
                (imlp_hbm, iq_v, il_v, pk_d, False))

        def fire(step):
            tbl, q_v, _, _, _ = plan[step // NGCH]
            c = step % NGCH
            p = step % 2
            return pltpu.async_copy(
                tbl.at[q_v.at[pl.ds(c * GCH, GCH)]], gbuf.at[p], sems[p])

        def extract(step):
            _, _, l_v, pk, fuse = plan[step // NGCH]
            c = step % NGCH
            p = step % 2

            def body(g2, carry):
                base = c * GCH + g2 * 16
                lvec = l_v[pl.ds(base, 16)]
                for k in range(16):
                    lane = lvec[k]
                    j = g2 * 16 + k
                    q = c * (GCH // 4) + j // 4
                    od = (j % 4) * D
                    v0 = gbuf[p, j, pl.ds(lane, 16)]
                    v1 = gbuf[p, j, pl.ds(lane + 16, 16)]
                    if fuse:
                        pk[q, pl.ds(od, 16)] = pk[q, pl.ds(od, 16)] * v0
                        pk[q, pl.ds(od + 16, 16)] = (
                            pk[q, pl.ds(od + 16, 16)] * v1)
                    else:
                        pk[q, pl.ds(od, 16)] = v0
                        pk[q, pl.ds(od + 16, 16)] = v1
                return carry

            lax.fori_loop(0, GCH // 16, body, 0)

        nsteps = 4 * NGCH
        cp = fire(0)
        for step in range(nsteps):
            nxt = fire(step + 1) if step + 1 < nsteps else None
            cp.wait()
            extract(step)
            cp = nxt

        pltpu.sync_copy(pk_a, mf_out.at[pl.ds(pbase, PK)])
        pltpu.sync_copy(pk_c, umlp_out.at[pl.ds(pbase, PK)])
        pltpu.sync_copy(pk_d, imlp_out.at[pl.ds(pbase, PK)])

    return _sc_gather


BT = 1024


def _dense_body(mf_ref, umlp_ref, imlp_ref, b1u_ref, b1i_ref, b1_ref,
                b2w_ref, b2_ref, b3w_ref, b3_ref, bpmf_ref, bph_ref, bp_ref,
                out_ref):
    u = umlp_ref[...]
    it = imlp_ref[...]
    h = u @ b1u_ref[...] + it @ b1i_ref[...] + b1_ref[...]
    h = jnp.maximum(h, 0.0)
    h = jnp.maximum(h @ b2w_ref[...] + b2_ref[...], 0.0)
    h = h @ b3w_ref[...] + b3_ref[...]
    out_ref[...] = (mf_ref[...] @ bpmf_ref[...] + h @ bph_ref[...]
                    + bp_ref[...])


def _dense(mf, umlp, imlp, b1u, b1i, b1t, b2w, b2t, b3w, b3t, bpmf, bph, bp1):
    grid = ((B // 4) // BT,)
    row_spec = pl.BlockSpec((BT, 128), lambda i: (i, 0))
    full = lambda shape: pl.BlockSpec(shape, lambda i: (0,) * len(shape))
    return pl.pallas_call(
        _dense_body,
        grid=grid,
        in_specs=[
            row_spec, row_spec, row_spec,
            full((128, 256)), full((128, 256)), full((1, 256)),
            full((256, 128)), full((1, 128)),
            full((128, 64)), full((1, 64)),
            full((128, 4)), full((64, 4)), full((1, 1)),
        ],
        out_specs=pl.BlockSpec((BT, 4), lambda i: (i, 0)),
        out_shape=jax.ShapeDtypeStruct((B // 4, 4), jnp.float32),
    )(mf, umlp, imlp, b1u, b1i, b1t, b2w, b2t, b3w, b3t, bpmf, bph, bp1)


def kernel(user_indices, item_indices, embed_user_mf, embed_item_mf,
           embed_user_mlp, embed_item_mlp, W1, b1, W2, b2, W3, b3, Wp, bp):
    ui = user_indices.astype(jnp.int32)
    ii = item_indices.astype(jnp.int32)
    uq = (ui // 4).reshape(NW, BPW)
    iq = (ii // 4).reshape(NW, BPW)
    ul = ((ui % 4) * D).reshape(NW, BPW)
    il = ((ii % 4) * D).reshape(NW, BPW)
    r0, r1, r2, r3 = _build_repack()(
        embed_user_mf, embed_item_mf, embed_user_mlp, embed_item_mlp)
    mfp, umlp_p, imlp_p = _build_sc_gather()(uq, iq, ul, il, r0, r1, r2, r3)

    eye4 = jnp.eye(4, dtype=jnp.float32)
    b1u = jnp.kron(eye4, W1[:D])
    b1i = jnp.kron(eye4, W1[D:])
    b2w = jnp.kron(eye4, W2)
    b3w = jnp.kron(eye4, W3)
    bpmf = jnp.kron(eye4, Wp[:D])
    bph = jnp.kron(eye4, Wp[D:])
    out4 = _dense(
        mfp, umlp_p, imlp_p,
        b1u, b1i, jnp.tile(b1, 4).reshape(1, 256),
        b2w, jnp.tile(b2, 4).reshape(1, 128),
        b3w, jnp.tile(b3, 4).reshape(1, 64),
        bpmf, bph, bp.reshape(1, 1))
    return out4.reshape(B)


# speedup vs baseline: 1.9161x; 1.9161x over previous
import functools

import jax
import jax.numpy as jnp
from jax import lax
from jax.experimental import pallas as pl
from jax.experimental.pallas import tpu as pltpu
from jax.experimental.pallas import tpu_sc as plsc

B = 16384
D = 32
NR = 1000000
NW = 32
BPW = B // NW
PK = BPW // 4
CHB = 32
NCHK = BPW // CHB


@functools.cache
def _build_sc_gather():
    mesh = plsc.VectorSubcoreMesh(core_axis_name="c", subcore_axis_name="s")

    @functools.partial(
        pl.kernel,
        mesh=mesh,
        out_type=(
            jax.ShapeDtypeStruct((B // 4, 128), jnp.float32),
            jax.ShapeDtypeStruct((B // 4, 128), jnp.float32),
            jax.ShapeDtypeStruct((B // 4, 128), jnp.float32),
        ),
        scratch_types=(
            pltpu.VMEM((BPW,), jnp.int32),
            pltpu.VMEM((BPW,), jnp.int32),
            pltpu.VMEM((PK, 128), jnp.float32),
            pltpu.VMEM((PK, 128), jnp.float32),
            pltpu.VMEM((PK, 128), jnp.float32),
            pltpu.VMEM((PK, 128), jnp.float32),
            pltpu.SemaphoreType.DMA,
        ),
    )
    def _sc_gather(uidx_hbm, iidx_hbm,
                   umf_hbm, imf_hbm, umlp_hbm, imlp_hbm,
                   mf_out, umlp_out, imlp_out,
                   uidx_v, iidx_v, pk_a, pk_b, pk_c, pk_d,
                   sem):
        wid = lax.axis_index("s") * 2 + lax.axis_index("c")
        pbase = wid * PK

        pltpu.sync_copy(uidx_hbm.at[wid], uidx_v)
        pltpu.sync_copy(iidx_hbm.at[wid], iidx_v)

        def fire_into(tbl, idx_v, pk):
            def body(g, carry):
                vec = idx_v[pl.ds(g * 16, 16)]
                for k in range(16):
                    r = vec[k]
                    q = g * 4 + k // 4
                    lane = (k % 4) * D
                    pltpu.async_copy(tbl.at[r], pk.at[q, pl.ds(lane, D)],
                                     sem)
                return carry
            lax.fori_loop(0, BPW // 16, body, 0)

        fire_into(umf_hbm, uidx_v, pk_a)
        fire_into(imf_hbm, iidx_v, pk_b)
        fire_into(umlp_hbm, uidx_v, pk_c)
        fire_into(imlp_hbm, iidx_v, pk_d)

        for _ in range(4):
            pltpu.make_async_copy(
                mf_out.at[pl.ds(0, PK)], pk_a, sem).wait()

        def prod(i, carry):
            for h in range(8):
                s = pl.ds(h * 16, 16)
                pk_a[i, s] = pk_a[i, s] * pk_b[i, s]
            return carry
        lax.fori_loop(0, PK, prod, 0)

        pltpu.sync_copy(pk_a, mf_out.at[pl.ds(pbase, PK)])
        pltpu.sync_copy(pk_c, umlp_out.at[pl.ds(pbase, PK)])
        pltpu.sync_copy(pk_d, imlp_out.at[pl.ds(pbase, PK)])

    return _sc_gather


BT = 1024


def _dense_body(mf_ref, umlp_ref, imlp_ref, b1u_ref, b1i_ref, b1_ref,
                b2w_ref, b2_ref, b3w_ref, b3_ref, bpmf_ref, bph_ref, bp_ref,
                out_ref):
    u = umlp_ref[...]
    it = imlp_ref[...]
    h = u @ b1u_ref[...] + it @ b1i_ref[...] + b1_ref[...]
    h = jnp.maximum(h, 0.0)
    h = jnp.maximum(h @ b2w_ref[...] + b2_ref[...], 0.0)
    h = h @ b3w_ref[...] + b3_ref[...]
    out_ref[...] = (mf_ref[...] @ bpmf_ref[...] + h @ bph_ref[...]
                    + bp_ref[...])


def _dense(mf, umlp, imlp, b1u, b1i, b1t, b2w, b2t, b3w, b3t, bpmf, bph, bp1):
    grid = ((B // 4) // BT,)
    row_spec = pl.BlockSpec((BT, 128), lambda i: (i, 0))
    full = lambda shape: pl.BlockSpec(shape, lambda i: (0,) * len(shape))
    return pl.pallas_call(
        _dense_body,
        grid=grid,
        in_specs=[
            row_spec, row_spec, row_spec,
            full((128, 256)), full((128, 256)), full((1, 256)),
            full((256, 128)), full((1, 128)),
            full((128, 64)), full((1, 64)),
            full((128, 4)), full((64, 4)), full((1, 1)),
        ],
        out_specs=pl.BlockSpec((BT, 4), lambda i: (i, 0)),
        out_shape=jax.ShapeDtypeStruct((B // 4, 4), jnp.float32),
    )(mf, umlp, imlp, b1u, b1i, b1t, b2w, b2t, b3w, b3t, bpmf, bph, bp1)


def kernel(user_indices, item_indices, embed_user_mf, embed_item_mf,
           embed_user_mlp, embed_item_mlp, W1, b1, W2, b2, W3, b3, Wp, bp):
    ui = user_indices.astype(jnp.int32).reshape(NW, BPW)
    ii = item_indices.astype(jnp.int32).reshape(NW, BPW)
    mfp, umlp_p, imlp_p = _build_sc_gather()(
        ui, ii, embed_user_mf, embed_item_mf, embed_user_mlp, embed_item_mlp)

    eye4 = jnp.eye(4, dtype=jnp.float32)
    b1u = jnp.kron(eye4, W1[:D])
    b1i = jnp.kron(eye4, W1[D:])
    b2w = jnp.kron(eye4, W2)
    b3w = jnp.kron(eye4, W3)
    bpmf = jnp.kron(eye4, Wp[:D])
    bph = jnp.kron(eye4, Wp[D:])
    out4 = _dense(
        mfp, umlp_p, imlp_p,
        b1u, b1i, jnp.tile(b1, 4).reshape(1, 256),
        b2w, jnp.tile(b2, 4).reshape(1, 128),
        b3w, jnp.tile(b3, 4).reshape(1, 64),
        bpmf, bph, bp.reshape(1, 1))
    return out4.reshape(B)
